# Initial kernel scaffold; baseline (speedup 1.0000x reference)
import functools

import jax
import jax.numpy as jnp
from jax import lax
from jax.experimental import pallas as pl
from jax.experimental.pallas import tpu as pltpu
from jax.experimental.pallas import tpu_sc as plsc

RES = 512
DIM = 16
N_PTS = 2097152
NC, NS, LANES = 2, 16, 16
NW = NC * NS
PTS_W = N_PTS // NW
B = 128
STEPS = PTS_W // B
GROUPS = B // LANES
NTAPS = 12


def _pix(c):
    p = jnp.clip(c * 128.0 + 255.5, 0.0, float(RES - 1))
    i0 = p.astype(jnp.int32)
    f = p - i0.astype(jnp.float32)
    i1 = jnp.minimum(i0 + 1, RES - 1)
    return i0, i1, f


def _tec_body(xs, ys, zs, table, out, xv, yv, zv, idxv, wv, gath, obuf, sem):
    wid = lax.axis_index("s") * NC + lax.axis_index("c")

    def step_fn(step, _):
        base = wid * PTS_W + step * B
        pltpu.sync_copy(xs.at[pl.ds(base, B)], xv)
        pltpu.sync_copy(ys.at[pl.ds(base, B)], yv)
        pltpu.sync_copy(zs.at[pl.ds(base, B)], zv)

        for g in range(GROUPS):
            sl = pl.ds(g * LANES, LANES)
            x = xv[sl]
            y = yv[sl]
            z = zv[sl]
            xi0, xi1, xf = _pix(x)
            yi0, yi1, yf = _pix(y)
            zi0, zi1, zf = _pix(z)
            specs = ((xi0, xi1, xf, yi0, yi1, yf),
                     (yi0, yi1, yf, zi0, zi1, zf),
                     (xi0, xi1, xf, zi0, zi1, zf))
            for p, (ci0, ci1, cf, ri0, ri1, rf) in enumerate(specs):
                r0 = (p * RES * RES) + ri0 * RES
                r1 = r0 + (ri1 - ri0) * RES
                cw0 = 1.0 - cf
                rw0 = 1.0 - rf
                idxv[4 * p + 0, sl] = r0 + ci0
                idxv[4 * p + 1, sl] = r0 + ci1
                idxv[4 * p + 2, sl] = r1 + ci0
                idxv[4 * p + 3, sl] = r1 + ci1
                wv[4 * p + 0, sl] = rw0 * cw0
                wv[4 * p + 1, sl] = rw0 * cf
                wv[4 * p + 2, sl] = rf * cw0
                wv[4 * p + 3, sl] = rf * cf

        copies = [pltpu.async_copy(table.at[idxv.at[t]], gath.at[t], sem)
                  for t in range(NTAPS)]
        for c in copies:
            c.wait()

        def point_fn(p, _):
            for pi in range(3):
                t = 4 * pi
                acc = gath[t, p] * wv[t, p]
                acc = acc + gath[t + 1, p] * wv[t + 1, p]
                acc = acc + gath[t + 2, p] * wv[t + 2, p]
                acc = acc + gath[t + 3, p] * wv[t + 3, p]
                obuf[p, pl.ds(DIM * pi, DIM)] = acc
            return 0

        lax.fori_loop(0, B, point_fn, 0)
        pltpu.sync_copy(obuf, out.at[pl.ds(base, B)])
        return 0

    lax.fori_loop(0, STEPS, step_fn, 0)


@jax.jit
def _sc_sample(xs, ys, zs, table):
    mesh = plsc.VectorSubcoreMesh(core_axis_name="c", subcore_axis_name="s",
                                  num_cores=NC, num_subcores=NS)
    f = pl.kernel(
        _tec_body,
        out_type=jax.ShapeDtypeStruct((N_PTS, 3 * DIM), jnp.float32),
        mesh=mesh,
        scratch_types=[
            pltpu.VMEM((B,), jnp.float32),
            pltpu.VMEM((B,), jnp.float32),
            pltpu.VMEM((B,), jnp.float32),
            pltpu.VMEM((NTAPS, B), jnp.int32),
            pltpu.VMEM((NTAPS, B), jnp.float32),
            pltpu.VMEM((NTAPS, B, DIM), jnp.float32),
            pltpu.VMEM((B, 3 * DIM), jnp.float32),
            pltpu.SemaphoreType.DMA,
        ],
    )
    return f(xs, ys, zs, table)


def kernel(loc, triplane):
    locT = jnp.transpose(loc)
    table = jnp.transpose(triplane, (0, 2, 3, 1))
    table = table.reshape(3 * RES * RES, DIM)
    return _sc_sample(locT[0], locT[1], locT[2], table)


# speedup vs baseline: 64.8640x; 64.8640x over previous
import functools

import jax
import jax.numpy as jnp
from jax import lax
from jax.experimental import pallas as pl
from jax.experimental.pallas import tpu as pltpu
from jax.experimental.pallas import tpu_sc as plsc

RES = 512
DIM = 16
N_PTS = 2097152
NC, NS, LANES = 2, 16, 16
NW = NC * NS
PTS_W = N_PTS // NW
B = 128
STEPS = PTS_W // B
GROUPS = B // LANES
NTAPS = 12


def _pix(c):
    p = jnp.clip(c * 128.0 + 255.5, 0.0, float(RES - 1))
    i0 = p.astype(jnp.int32)
    f = p - i0.astype(jnp.float32)
    i1 = jnp.minimum(i0 + 1, RES - 1)
    return i0, i1, f


def _tec_body(xs, ys, zs, table, out, xv, yv, zv, idxv, wv, gath, obuf, sem):
    wid = lax.axis_index("s") * NC + lax.axis_index("c")

    def step_fn(step, _):
        base = wid * PTS_W + step * B
        pltpu.sync_copy(xs.at[pl.ds(base, B)], xv)
        pltpu.sync_copy(ys.at[pl.ds(base, B)], yv)
        pltpu.sync_copy(zs.at[pl.ds(base, B)], zv)

        for g in range(GROUPS):
            sl = pl.ds(g * LANES, LANES)
            x = xv[sl]
            y = yv[sl]
            z = zv[sl]
            xi0, xi1, xf = _pix(x)
            yi0, yi1, yf = _pix(y)
            zi0, zi1, zf = _pix(z)
            specs = ((xi0, xi1, xf, yi0, yi1, yf),
                     (yi0, yi1, yf, zi0, zi1, zf),
                     (xi0, xi1, xf, zi0, zi1, zf))
            for p, (ci0, ci1, cf, ri0, ri1, rf) in enumerate(specs):
                r0 = (p * RES * RES) + ri0 * RES
                r1 = r0 + (ri1 - ri0) * RES
                cw0 = 1.0 - cf
                rw0 = 1.0 - rf
                idxv[4 * p + 0, sl] = r0 + ci0
                idxv[4 * p + 1, sl] = r0 + ci1
                idxv[4 * p + 2, sl] = r1 + ci0
                idxv[4 * p + 3, sl] = r1 + ci1
                wv[4 * p + 0, sl] = rw0 * cw0
                wv[4 * p + 1, sl] = rw0 * cf
                wv[4 * p + 2, sl] = rf * cw0
                wv[4 * p + 3, sl] = rf * cf

        copies = [pltpu.async_copy(table.at[idxv.at[t]],
                                   gath.at[pl.ds(t * B, B)], sem)
                  for t in range(NTAPS)]
        for c in copies:
            c.wait()

        def wsum_fn(g, _):
            sl = pl.ds(g * LANES, LANES)
            pid = lax.iota(jnp.int32, LANES) + g * LANES
            for pi in range(3):
                w = [wv[4 * pi + t, sl] for t in range(4)]
                row = [(4 * pi + t) * B + pid for t in range(4)]
                for c in range(DIM):
                    cv = jnp.full((LANES,), c, jnp.int32)
                    acc = plsc.load_gather(gath, (row[0], cv)) * w[0]
                    acc += plsc.load_gather(gath, (row[1], cv)) * w[1]
                    acc += plsc.load_gather(gath, (row[2], cv)) * w[2]
                    acc += plsc.load_gather(gath, (row[3], cv)) * w[3]
                    ov = pid * (3 * DIM) + (pi * DIM + c)
                    plsc.store_scatter(obuf, (ov,), acc)
            return 0

        lax.fori_loop(0, GROUPS, wsum_fn, 0)
        pltpu.sync_copy(obuf, out.at[pl.ds(base * (3 * DIM), B * 3 * DIM)])
        return 0

    lax.fori_loop(0, STEPS, step_fn, 0)


@jax.jit
def _sc_sample(xs, ys, zs, table):
    mesh = plsc.VectorSubcoreMesh(core_axis_name="c", subcore_axis_name="s",
                                  num_cores=NC, num_subcores=NS)
    f = pl.kernel(
        _tec_body,
        out_type=jax.ShapeDtypeStruct((N_PTS * 3 * DIM,), jnp.float32),
        mesh=mesh,
        scratch_types=[
            pltpu.VMEM((B,), jnp.float32),
            pltpu.VMEM((B,), jnp.float32),
            pltpu.VMEM((B,), jnp.float32),
            pltpu.VMEM((NTAPS, B), jnp.int32),
            pltpu.VMEM((NTAPS, B), jnp.float32),
            pltpu.VMEM((NTAPS * B, DIM), jnp.float32),
            pltpu.VMEM((B * 3 * DIM,), jnp.float32),
            pltpu.SemaphoreType.DMA,
        ],
        compiler_params=pltpu.CompilerParams(use_tc_tiling_on_sc=False,
                                             needs_layout_passes=False),
    )
    return f(xs, ys, zs, table)


def kernel(loc, triplane):
    locT = jnp.transpose(loc)
    table = jnp.transpose(triplane, (0, 2, 3, 1))
    table = table.reshape(3 * RES * RES, DIM)
    out = _sc_sample(locT[0], locT[1], locT[2], table)
    return out.reshape(N_PTS, 3 * DIM)


# speedup vs baseline: 81.2329x; 1.2524x over previous
import jax
import jax.numpy as jnp
from jax import lax
from jax.experimental import pallas as pl
from jax.experimental.pallas import tpu as pltpu
from jax.experimental.pallas import tpu_sc as plsc

RES = 512
DIM = 16
ODIM = 3 * DIM
N_PTS = 2097152
NC, NS, LANES = 2, 16, 16
NW = NC * NS
PTS_W = N_PTS // NW
B = 128
STEPS = PTS_W // B
HALF = STEPS // 2
GROUPS = B // LANES
NTAPS = 12


def _pix(c):
    p = jnp.clip(c * 128.0 + 255.5, 0.0, float(RES - 1))
    i0 = p.astype(jnp.int32)
    f = p - i0.astype(jnp.float32)
    i1 = jnp.minimum(i0 + 1, RES - 1)
    return i0, i1, f


def _tec_body(locT, table, out, cbuf, idxv, wv, gath, ob0, ob1,
              semc0, semc1, semg0, semg1, semo0, semo1):
    wid = lax.axis_index("s") * NC + lax.axis_index("c")
    semc = (semc0, semc1)
    semg = (semg0, semg1)
    semo = (semo0, semo1)
    obs = (ob0, ob1)

    def fire_coords(q, s):
        base = wid * PTS_W + s * B
        pltpu.async_copy(locT.at[:, pl.ds(base, B)], cbuf.at[q], semc[q])

    def wait_coords(q):
        pltpu.make_async_copy(locT.at[:, pl.ds(0, B)], cbuf.at[q],
                              semc[q]).wait()

    def compute_idx(q):
        for g in range(GROUPS):
            sl = pl.ds(g * LANES, LANES)
            x = cbuf[q, 0, sl]
            y = cbuf[q, 1, sl]
            z = cbuf[q, 2, sl]
            xi0, xi1, xf = _pix(x)
            yi0, yi1, yf = _pix(y)
            zi0, zi1, zf = _pix(z)
            specs = ((xi0, xi1, xf, yi0, yi1, yf),
                     (yi0, yi1, yf, zi0, zi1, zf),
                     (xi0, xi1, xf, zi0, zi1, zf))
            for p, (ci0, ci1, cf, ri0, ri1, rf) in enumerate(specs):
                r0 = (p * RES * RES) + ri0 * RES
                r1 = r0 + (ri1 - ri0) * RES
                cw0 = 1.0 - cf
                rw0 = 1.0 - rf
                o = q * NTAPS + 4 * p
                idxv[o + 0, sl] = r0 + ci0
                idxv[o + 1, sl] = r0 + ci1
                idxv[o + 2, sl] = r1 + ci0
                idxv[o + 3, sl] = r1 + ci1
                wv[o + 0, sl] = rw0 * cw0
                wv[o + 1, sl] = rw0 * cf
                wv[o + 2, sl] = rf * cw0
                wv[o + 3, sl] = rf * cf

    def fire_gathers(q):
        for t in range(NTAPS):
            r = q * NTAPS + t
            pltpu.async_copy(table.at[idxv.at[r]],
                             gath.at[pl.ds(r * B, B)], semg[q])

    def wait_gathers(q):
        for t in range(NTAPS):
            r = q * NTAPS + t
            pltpu.make_async_copy(table.at[idxv.at[r]],
                                  gath.at[pl.ds(r * B, B)], semg[q]).wait()

    def wsum(q):
        ob = obs[q]

        def wsum_fn(g, _):
            sl = pl.ds(g * LANES, LANES)
            pid = lax.iota(jnp.int32, LANES) + g * LANES
            for pi in range(3):
                w = [wv[q * NTAPS + 4 * pi + t, sl] for t in range(4)]
                row = [(q * NTAPS + 4 * pi + t) * B + pid for t in range(4)]
                for c in range(DIM):
                    cv = jnp.full((LANES,), c, jnp.int32)
                    acc = plsc.load_gather(gath, (row[0], cv)) * w[0]
                    acc += plsc.load_gather(gath, (row[1], cv)) * w[1]
                    acc += plsc.load_gather(gath, (row[2], cv)) * w[2]
                    acc += plsc.load_gather(gath, (row[3], cv)) * w[3]
                    ov = pid * ODIM + (pi * DIM + c)
                    plsc.store_scatter(ob, (ov,), acc)
            return 0

        lax.fori_loop(0, GROUPS, wsum_fn, 0)

    def fire_out(q, s):
        base = (wid * PTS_W + s * B) * ODIM
        pltpu.async_copy(obs[q], out.at[pl.ds(base, B * ODIM)], semo[q])

    def wait_out(q):
        pltpu.make_async_copy(obs[q], out.at[pl.ds(0, B * ODIM)],
                              semo[q]).wait()

    fire_coords(0, 0)
    wait_coords(0)
    compute_idx(0)
    fire_gathers(0)
    fire_coords(1, 1)
    wait_coords(1)
    compute_idx(1)
    fire_gathers(1)
    fire_coords(0, 2)
    wait_gathers(0)
    wsum(0)
    fire_out(0, 0)

    def pair_body(i2, _):
        a = 2 * i2 + 1
        wait_coords(0)
        compute_idx(0)
        fire_gathers(0)
        fire_coords(1, a + 2)
        wait_gathers(1)

        @pl.when(i2 >= 1)
        def _():
            wait_out(1)

        wsum(1)
        fire_out(1, a)

        b = a + 1
        wait_coords(1)
        compute_idx(1)
        fire_gathers(1)

        @pl.when(i2 < HALF - 2)
        def _():
            fire_coords(0, b + 2)

        wait_gathers(0)
        wait_out(0)
        wsum(0)
        fire_out(0, b)
        return 0

    lax.fori_loop(0, HALF - 1, pair_body, 0)

    wait_gathers(1)
    wait_out(1)
    wsum(1)
    fire_out(1, STEPS - 1)
    wait_out(0)
    wait_out(1)


@jax.jit
def _sc_sample(locT, table):
    mesh = plsc.VectorSubcoreMesh(core_axis_name="c", subcore_axis_name="s",
                                  num_cores=NC, num_subcores=NS)
    f = pl.kernel(
        _tec_body,
        out_type=jax.ShapeDtypeStruct((N_PTS * ODIM,), jnp.float32),
        mesh=mesh,
        scratch_types=[
            pltpu.VMEM((2, 3, B), jnp.float32),
            pltpu.VMEM((2 * NTAPS, B), jnp.int32),
            pltpu.VMEM((2 * NTAPS, B), jnp.float32),
            pltpu.VMEM((2 * NTAPS * B, DIM), jnp.float32),
            pltpu.VMEM((B * ODIM,), jnp.float32),
            pltpu.VMEM((B * ODIM,), jnp.float32),
            pltpu.SemaphoreType.DMA,
            pltpu.SemaphoreType.DMA,
            pltpu.SemaphoreType.DMA,
            pltpu.SemaphoreType.DMA,
            pltpu.SemaphoreType.DMA,
            pltpu.SemaphoreType.DMA,
        ],
        compiler_params=pltpu.CompilerParams(use_tc_tiling_on_sc=False,
                                             needs_layout_passes=False),
    )
    return f(locT, table)


def kernel(loc, triplane):
    locT = jnp.transpose(loc)
    table = jnp.transpose(triplane, (0, 2, 3, 1))
    table = table.reshape(3 * RES * RES, DIM)
    out = _sc_sample(locT, table)
    return out.reshape(N_PTS, ODIM)


# speedup vs baseline: 83.6804x; 1.0301x over previous
import jax
import jax.numpy as jnp
from jax import lax
from jax.experimental import pallas as pl
from jax.experimental.pallas import tpu as pltpu
from jax.experimental.pallas import tpu_sc as plsc

RES = 512
DIM = 16
ODIM = 3 * DIM
QW = 32
N_PTS = 2097152
NC, NS, LANES = 2, 16, 16
NW = NC * NS
PTS_W = N_PTS // NW
B = 128
STEPS = PTS_W // B
HALF = STEPS // 2
GROUPS = B // LANES


def _pix(c):
    p = jnp.clip(c * 128.0 + 255.5, 0.0, float(RES - 1))
    i0 = p.astype(jnp.int32)
    f = p - i0.astype(jnp.float32)
    return i0, f


def _tec_body(locT, table, out, cbuf, idxv, wv, gath, ob0, ob1,
              semc0, semc1, semg0, semg1, semo0, semo1):
    wid = lax.axis_index("s") * NC + lax.axis_index("c")
    semc = (semc0, semc1)
    semg = (semg0, semg1)
    semo = (semo0, semo1)
    obs = (ob0, ob1)

    def fire_coords(q, s):
        base = wid * PTS_W + s * B
        pltpu.async_copy(locT.at[:, pl.ds(base, B)], cbuf.at[q], semc[q])

    def wait_coords(q):
        pltpu.make_async_copy(locT.at[:, pl.ds(0, B)], cbuf.at[q],
                              semc[q]).wait()

    def compute_idx(q):
        for g in range(GROUPS):
            sl = pl.ds(g * LANES, LANES)
            x = cbuf[q, 0, sl]
            y = cbuf[q, 1, sl]
            z = cbuf[q, 2, sl]
            xi0, xf = _pix(x)
            yi0, yf = _pix(y)
            zi0, zf = _pix(z)
            specs = ((xi0, xf, yi0, yf),
                     (yi0, yf, zi0, zf),
                     (xi0, xf, zi0, zf))
            for p, (ci0, cf, ri0, rf) in enumerate(specs):
                idxv[q * 3 + p, sl] = (p * RES * RES) + ri0 * RES + ci0
                cw0 = 1.0 - cf
                rw0 = 1.0 - rf
                o = q * 12 + 4 * p
                wv[o + 0, sl] = rw0 * cw0
                wv[o + 1, sl] = rw0 * cf
                wv[o + 2, sl] = rf * cw0
                wv[o + 3, sl] = rf * cf

    def fire_gathers(q):
        for p in range(3):
            r = q * 3 + p
            pltpu.async_copy(table.at[idxv.at[r]],
                             gath.at[pl.ds(r * B, B)], semg[q])

    def wait_gathers(q):
        for p in range(3):
            r = q * 3 + p
            pltpu.make_async_copy(table.at[idxv.at[r]],
                                  gath.at[pl.ds(r * B, B)], semg[q]).wait()

    def wsum(q):
        ob = obs[q]

        def wsum_fn(g, _):
            sl = pl.ds(g * LANES, LANES)
            pid = lax.iota(jnp.int32, LANES) + g * LANES
            for pi in range(3):
                w = [wv[q * 12 + 4 * pi + t, sl] for t in range(4)]
                row = (q * 3 + pi) * B + pid
                for cp in range(DIM // 2):
                    av = []
                    bv = []
                    for t in range(4):
                        cv = jnp.full((LANES,), t * (DIM // 2) + cp, jnp.int32)
                        v = plsc.load_gather(gath, (row, cv))
                        a, b = plsc.unpack(
                            plsc.bitcast(v, jnp.bfloat16),
                            format=plsc.PackFormat.INTERLEAVED)
                        av.append(a)
                        bv.append(b)
                    acca = av[0] * w[0] + av[1] * w[1] + av[2] * w[2] \
                        + av[3] * w[3]
                    accb = bv[0] * w[0] + bv[1] * w[1] + bv[2] * w[2] \
                        + bv[3] * w[3]
                    ov = pid * ODIM + (pi * DIM + 2 * cp)
                    plsc.store_scatter(ob, (ov,), acca)
                    plsc.store_scatter(ob, (ov + 1,), accb)
            return 0

        lax.fori_loop(0, GROUPS, wsum_fn, 0)

    def fire_out(q, s):
        base = (wid * PTS_W + s * B) * ODIM
        pltpu.async_copy(obs[q], out.at[pl.ds(base, B * ODIM)], semo[q])

    def wait_out(q):
        pltpu.make_async_copy(obs[q], out.at[pl.ds(0, B * ODIM)],
                              semo[q]).wait()

    fire_coords(0, 0)
    wait_coords(0)
    compute_idx(0)
    fire_gathers(0)
    fire_coords(1, 1)
    wait_coords(1)
    compute_idx(1)
    fire_gathers(1)
    fire_coords(0, 2)
    wait_gathers(0)
    wsum(0)
    fire_out(0, 0)

    def pair_body(i2, _):
        a = 2 * i2 + 1
        wait_coords(0)
        compute_idx(0)
        fire_gathers(0)
        fire_coords(1, a + 2)
        wait_gathers(1)

        @pl.when(i2 >= 1)
        def _():
            wait_out(1)

        wsum(1)
        fire_out(1, a)

        b = a + 1
        wait_coords(1)
        compute_idx(1)
        fire_gathers(1)

        @pl.when(i2 < HALF - 2)
        def _():
            fire_coords(0, b + 2)

        wait_gathers(0)
        wait_out(0)
        wsum(0)
        fire_out(0, b)
        return 0

    lax.fori_loop(0, HALF - 1, pair_body, 0)

    wait_gathers(1)
    wait_out(1)
    wsum(1)
    fire_out(1, STEPS - 1)
    wait_out(0)
    wait_out(1)


@jax.jit
def _sc_sample(locT, table):
    mesh = plsc.VectorSubcoreMesh(core_axis_name="c", subcore_axis_name="s",
                                  num_cores=NC, num_subcores=NS)
    f = pl.kernel(
        _tec_body,
        out_type=jax.ShapeDtypeStruct((N_PTS * ODIM,), jnp.float32),
        mesh=mesh,
        scratch_types=[
            pltpu.VMEM((2, 3, B), jnp.float32),
            pltpu.VMEM((2 * 3, B), jnp.int32),
            pltpu.VMEM((2 * 12, B), jnp.float32),
            pltpu.VMEM((2 * 3 * B, QW), jnp.int32),
            pltpu.VMEM((B * ODIM,), jnp.float32),
            pltpu.VMEM((B * ODIM,), jnp.float32),
            pltpu.SemaphoreType.DMA,
            pltpu.SemaphoreType.DMA,
            pltpu.SemaphoreType.DMA,
            pltpu.SemaphoreType.DMA,
            pltpu.SemaphoreType.DMA,
            pltpu.SemaphoreType.DMA,
        ],
        compiler_params=pltpu.CompilerParams(use_tc_tiling_on_sc=False,
                                             needs_layout_passes=False),
    )
    return f(locT, table)


def kernel(loc, triplane):
    locT = jnp.transpose(loc)
    t = jnp.transpose(triplane, (0, 2, 3, 1))
    t = t.astype(jnp.bfloat16)
    tx = jnp.concatenate([t[:, :, 1:], t[:, :, -1:]], axis=2)
    ty = jnp.concatenate([t[:, 1:], t[:, -1:]], axis=1)
    txy = jnp.concatenate([tx[:, 1:], tx[:, -1:]], axis=1)
    tq = jnp.concatenate([t, tx, ty, txy], axis=3)
    tq = jax.lax.bitcast_convert_type(
        tq.reshape(3 * RES * RES, QW, 2), jnp.int32)
    out = _sc_sample(locT, tq)
    return out.reshape(N_PTS, ODIM)
